# baseline (speedup 1.0000x reference)
import functools

import jax
import jax.numpy as jnp
from jax import lax
from jax.experimental import pallas as pl
from jax.experimental.pallas import tpu as pltpu
from jax.experimental.pallas import tpu_sc as plsc

_NC = 2
_NS = 16
_NW = _NC * _NS
_CH = 128


def _mesh():
    return plsc.VectorSubcoreMesh(
        core_axis_name="c", subcore_axis_name="s",
        num_cores=_NC, num_subcores=_NS)


def _sc_degree(n, nacc, cpw, epw):
    seg = nacc // _NS
    nv = epw // 16

    @functools.partial(
        pl.kernel, mesh=_mesh(),
        out_type=jax.ShapeDtypeStruct((_NC, nacc), jnp.float32),
        compiler_params=pltpu.CompilerParams(use_tc_tiling_on_sc=False),
        scratch_types=[
            pltpu.VMEM((epw,), jnp.int32),
            pltpu.VMEM((cpw, _CH), jnp.int32),
            pltpu.VMEM((_CH,), jnp.float32),
            pltpu.VMEM_SHARED((nacc,), jnp.float32),
            pltpu.SemaphoreType.DMA,
        ],
    )
    def deg_kernel(ei_hbm, zeros_hbm, ones_hbm, out_hbm, dtmp, didx, ones_v,
                   acc, sem):
        c = lax.axis_index("c")
        s = lax.axis_index("s")
        wid = s * _NC + c
        pltpu.sync_copy(ei_hbm.at[1, pl.ds(wid * epw, epw)], dtmp)
        pltpu.sync_copy(ones_hbm, ones_v)

        @pl.loop(0, nv, unroll=8)
        def _(i):
            didx[i // 8, pl.ds((i % 8) * 16, 16)] = dtmp[pl.ds(i * 16, 16)]

        for k in range(cpw * 8 - nv):
            g = nv + k
            didx[g // 8, pl.ds((g % 8) * 16, 16)] = jnp.full(
                (16,), n + k, jnp.int32)
        pltpu.sync_copy(zeros_hbm.at[pl.ds(s * seg, seg)],
                        acc.at[pl.ds(s * seg, seg)])
        plsc.subcore_barrier()

        @pl.loop(0, cpw)
        def _(j):
            pltpu.async_copy(ones_v, acc.at[didx.at[j]], sem, add=True)

        @pl.loop(0, cpw)
        def _(j):
            pltpu.make_async_copy(ones_v, acc.at[didx.at[0]], sem).wait()

        plsc.subcore_barrier()
        pltpu.sync_copy(acc.at[pl.ds(s * seg, seg)],
                        out_hbm.at[c, pl.ds(s * seg, seg)])

    return deg_kernel


def _sc_propagate(n, nacc, cpw, d):
    seg = nacc // _NS
    gseg = n // _NS

    @functools.partial(
        pl.kernel, mesh=_mesh(),
        out_type=jax.ShapeDtypeStruct((_NC, nacc, d), jnp.float32),
        compiler_params=pltpu.CompilerParams(use_tc_tiling_on_sc=False),
        scratch_types=[
            pltpu.VMEM((cpw, _CH), jnp.int32),
            pltpu.VMEM((cpw, _CH), jnp.int32),
            pltpu.VMEM((4, _CH, d), jnp.float32),
            pltpu.VMEM_SHARED((nacc, d), jnp.float32),
            pltpu.VMEM_SHARED((n, d), jnp.float32),
            [pltpu.SemaphoreType.DMA] * 4,
            [pltpu.SemaphoreType.DMA] * 4,
        ],
    )
    def prop_kernel(g_hbm, edges_hbm, zeros_hbm, out_hbm,
                    sidx, didx, bufs, acc, g_sh, gsem, ssem):
        c = lax.axis_index("c")
        s = lax.axis_index("s")
        wid = s * _NC + c
        pltpu.sync_copy(edges_hbm.at[0, wid], sidx)
        pltpu.sync_copy(edges_hbm.at[1, wid], didx)
        pltpu.sync_copy(zeros_hbm.at[pl.ds(s * seg, seg), :],
                        acc.at[pl.ds(s * seg, seg), :])
        pltpu.sync_copy(g_hbm.at[pl.ds(s * gseg, gseg), :],
                        g_sh.at[pl.ds(s * gseg, gseg), :])
        plsc.subcore_barrier()

        for b in range(3):
            pltpu.async_copy(g_sh.at[sidx.at[b]], bufs.at[b], gsem[b])

        @pl.loop(0, cpw, step=4)
        def _(j):
            for b in range(4):
                jb = j + b
                ns = (b + 3) % 4

                @pl.when(jb > 0)
                def _():
                    pltpu.make_async_copy(
                        bufs.at[ns], acc.at[didx.at[0]], ssem[ns]).wait()

                @pl.when(jb + 3 < cpw)
                def _():
                    pltpu.async_copy(
                        g_sh.at[sidx.at[jb + 3]], bufs.at[ns], gsem[ns])

                pltpu.make_async_copy(
                    g_sh.at[sidx.at[jb]], bufs.at[b], gsem[b]).wait()
                pltpu.async_copy(
                    bufs.at[b], acc.at[didx.at[jb]], ssem[b], add=True)

        pltpu.make_async_copy(bufs.at[3], acc.at[didx.at[0]], ssem[3]).wait()
        plsc.subcore_barrier()
        pltpu.sync_copy(acc.at[pl.ds(s * seg, seg), :],
                        out_hbm.at[c, pl.ds(s * seg, seg), :])

    return prop_kernel


def _tc_matmul(xr, w0r):
    nr = xr.shape[0]
    dr = w0r.shape[1]

    def body(x_ref, w_ref, xw_ref):
        xw_ref[...] = jnp.dot(x_ref[...], w_ref[...],
                              preferred_element_type=jnp.float32)

    return pl.pallas_call(
        body,
        out_shape=jax.ShapeDtypeStruct((nr, dr), jnp.float32),
    )(xr, w0r)


def _tc_rsqrt(degp2):
    nr, dr = degp2.shape[1:]

    def body(degp_ref, dinv_ref):
        dinv_ref[...] = lax.rsqrt(degp_ref[0] + degp_ref[1] + 1.0)

    return pl.pallas_call(
        body,
        out_shape=jax.ShapeDtypeStruct((nr, dr), jnp.float32),
    )(degp2)


def _tc_scale(dinvr, xwr):
    nr, dr = xwr.shape

    def body(dinv_ref, xw_ref, g0_ref):
        g0_ref[...] = xw_ref[...] * dinv_ref[...]

    return pl.pallas_call(
        body,
        out_shape=jax.ShapeDtypeStruct((nr, dr), jnp.float32),
    )(dinvr, xwr)


def _tc_mid(partsr, g0r, dinvr, b0t):
    nr, dr = g0r.shape

    def body(p_ref, g0_ref, dinv_ref, b_ref, g1_ref):
        ssum = p_ref[0, :nr, :] + p_ref[1, :nr, :] + g0_ref[...]
        h0 = jnp.maximum(ssum * dinv_ref[...] + b_ref[...], 0.0)
        g1_ref[...] = h0 * dinv_ref[...]

    return pl.pallas_call(
        body,
        out_shape=jax.ShapeDtypeStruct((nr, dr), jnp.float32),
    )(partsr, g0r, dinvr, b0t)


def _tc_out(partsr, g1r, dinvr, w1r, b1t):
    nr, dr = g1r.shape
    ncr = w1r.shape[1]

    def body(p_ref, g1_ref, dinv_ref, w_ref, b_ref, out_ref):
        ssum = (p_ref[0, :nr, :] + p_ref[1, :nr, :] + g1_ref[...]) * dinv_ref[...]
        out_ref[...] = jnp.dot(ssum, w_ref[...],
                               preferred_element_type=jnp.float32) + b_ref[...]

    return pl.pallas_call(
        body,
        out_shape=jax.ShapeDtypeStruct((nr, ncr), jnp.float32),
    )(partsr, g1r, dinvr, w1r, b1t)


def _block_diag(w, k):
    a, b = w.shape
    eye = jnp.eye(k, dtype=w.dtype)
    return (eye[:, None, :, None] * w[None, :, None, :]).reshape(k * a, k * b)


def kernel(x, edge_index, W0, b0, W1, b1):
    n = x.shape[1]
    e = edge_index.shape[2]
    dh = W0.shape[1]

    x2 = x[0]
    ei = edge_index[0].astype(jnp.int32)

    cpw = -(-e // (_NW * _CH))
    cpw += (-cpw) % 4
    epad = _NW * cpw * _CH
    npad_rows = 240
    nacc = n + npad_rows

    pad = epad - e
    pad_ar = jnp.arange(pad, dtype=jnp.int32)
    pads = jnp.stack([pad_ar % n, n + (pad_ar % npad_rows)])
    edges = jnp.concatenate([ei, pads], axis=1).reshape(2, _NW, cpw, _CH)

    ones_c1 = jnp.ones((_CH,), jnp.float32)
    zeros1 = jnp.zeros((nacc,), jnp.float32)
    zeros2 = jnp.zeros((nacc, dh), jnp.float32)

    pk = 128 // dh
    nr = n // pk
    nc = W1.shape[1]
    xr = x2.reshape(nr, pk * x2.shape[1])
    w0r = _block_diag(W0, pk)
    w1r = _block_diag(W1, pk)
    b0t = jnp.tile(b0, pk).reshape(1, pk * dh)
    b1t = jnp.tile(b1, pk).reshape(1, pk * nc)

    xwr = _tc_matmul(xr, w0r)
    degp = _sc_degree(n, nacc, cpw, e // _NW)(ei, zeros1, ones_c1)
    dinv1 = _tc_rsqrt(degp.reshape(_NC, nacc // 128, 128))
    dinvr = jnp.repeat(dinv1.reshape(-1)[:n], dh).reshape(nr, pk * dh)
    g0r = _tc_scale(dinvr, xwr)
    g0 = g0r.reshape(n, dh)
    parts1 = _sc_propagate(n, nacc, cpw, dh)(g0, edges, zeros2)
    g1r = _tc_mid(parts1.reshape(_NC, nacc // pk, pk * dh), g0r, dinvr, b0t)
    g1 = g1r.reshape(n, dh)
    parts2 = _sc_propagate(n, nacc, cpw, dh)(g1, edges, zeros2)
    outr = _tc_out(parts2.reshape(_NC, nacc // pk, pk * dh), g1r, dinvr,
                   w1r, b1t)
    return outr.reshape(n, nc)


# speedup vs baseline: 1.0604x; 1.0604x over previous
import functools

import jax
import jax.numpy as jnp
from jax import lax
from jax.experimental import pallas as pl
from jax.experimental.pallas import tpu as pltpu
from jax.experimental.pallas import tpu_sc as plsc

_NC = 2
_NS = 16
_NW = _NC * _NS
_CH = 128


def _mesh():
    return plsc.VectorSubcoreMesh(
        core_axis_name="c", subcore_axis_name="s",
        num_cores=_NC, num_subcores=_NS)


def _sc_degree(nacc, cpw):
    seg = nacc // _NS

    @functools.partial(
        pl.kernel, mesh=_mesh(),
        out_type=jax.ShapeDtypeStruct((_NC, nacc), jnp.float32),
        compiler_params=pltpu.CompilerParams(use_tc_tiling_on_sc=False),
        scratch_types=[
            pltpu.VMEM((cpw, _CH), jnp.int32),
            pltpu.VMEM((_CH,), jnp.float32),
            pltpu.VMEM_SHARED((nacc,), jnp.float32),
            pltpu.SemaphoreType.DMA,
        ],
    )
    def deg_kernel(edges_hbm, zeros_hbm, ones_hbm, out_hbm, didx, ones_v, acc,
                   sem):
        c = lax.axis_index("c")
        s = lax.axis_index("s")
        wid = s * _NC + c
        pltpu.sync_copy(edges_hbm.at[1, wid], didx)
        pltpu.sync_copy(ones_hbm, ones_v)
        pltpu.sync_copy(zeros_hbm.at[pl.ds(s * seg, seg)],
                        acc.at[pl.ds(s * seg, seg)])
        plsc.subcore_barrier()

        @pl.loop(0, cpw)
        def _(j):
            pltpu.async_copy(ones_v, acc.at[didx.at[j]], sem, add=True)

        @pl.loop(0, cpw)
        def _(j):
            pltpu.make_async_copy(ones_v, acc.at[didx.at[0]], sem).wait()

        plsc.subcore_barrier()
        pltpu.sync_copy(acc.at[pl.ds(s * seg, seg)],
                        out_hbm.at[c, pl.ds(s * seg, seg)])

    return deg_kernel


def _sc_propagate(n, nacc, cpw, d):
    seg = nacc // _NS
    gseg = n // _NS

    @functools.partial(
        pl.kernel, mesh=_mesh(),
        out_type=jax.ShapeDtypeStruct((_NC, nacc, d), jnp.float32),
        compiler_params=pltpu.CompilerParams(use_tc_tiling_on_sc=False),
        scratch_types=[
            pltpu.VMEM((cpw, _CH), jnp.int32),
            pltpu.VMEM((cpw, _CH), jnp.int32),
            pltpu.VMEM((4, _CH, d), jnp.float32),
            pltpu.VMEM_SHARED((nacc, d), jnp.float32),
            pltpu.VMEM_SHARED((n, d), jnp.float32),
            [pltpu.SemaphoreType.DMA] * 4,
            [pltpu.SemaphoreType.DMA] * 4,
        ],
    )
    def prop_kernel(g_hbm, edges_hbm, zeros_hbm, out_hbm,
                    sidx, didx, bufs, acc, g_sh, gsem, ssem):
        c = lax.axis_index("c")
        s = lax.axis_index("s")
        wid = s * _NC + c
        pltpu.sync_copy(edges_hbm.at[0, wid], sidx)
        pltpu.sync_copy(edges_hbm.at[1, wid], didx)
        pltpu.sync_copy(zeros_hbm.at[pl.ds(s * seg, seg), :],
                        acc.at[pl.ds(s * seg, seg), :])
        pltpu.sync_copy(g_hbm.at[pl.ds(s * gseg, gseg), :],
                        g_sh.at[pl.ds(s * gseg, gseg), :])
        plsc.subcore_barrier()

        for b in range(3):
            pltpu.async_copy(g_sh.at[sidx.at[b]], bufs.at[b], gsem[b])

        @pl.loop(0, cpw, step=4)
        def _(j):
            for b in range(4):
                jb = j + b
                ns = (b + 3) % 4

                @pl.when(jb > 0)
                def _():
                    pltpu.make_async_copy(
                        bufs.at[ns], acc.at[didx.at[0]], ssem[ns]).wait()

                @pl.when(jb + 3 < cpw)
                def _():
                    pltpu.async_copy(
                        g_sh.at[sidx.at[jb + 3]], bufs.at[ns], gsem[ns])

                pltpu.make_async_copy(
                    g_sh.at[sidx.at[jb]], bufs.at[b], gsem[b]).wait()
                pltpu.async_copy(
                    bufs.at[b], acc.at[didx.at[jb]], ssem[b], add=True)

        pltpu.make_async_copy(bufs.at[3], acc.at[didx.at[0]], ssem[3]).wait()
        plsc.subcore_barrier()
        pltpu.sync_copy(acc.at[pl.ds(s * seg, seg), :],
                        out_hbm.at[c, pl.ds(s * seg, seg), :])

    return prop_kernel


def _tc_matmul(xr, w0r):
    nr = xr.shape[0]
    dr = w0r.shape[1]

    def body(x_ref, w_ref, xw_ref):
        xw_ref[...] = jnp.dot(x_ref[...], w_ref[...],
                              preferred_element_type=jnp.float32)

    return pl.pallas_call(
        body,
        out_shape=jax.ShapeDtypeStruct((nr, dr), jnp.float32),
    )(xr, w0r)


def _tc_rsqrt(degp2):
    nr, dr = degp2.shape[1:]

    def body(degp_ref, dinv_ref):
        dinv_ref[...] = lax.rsqrt(degp_ref[0] + degp_ref[1] + 1.0)

    return pl.pallas_call(
        body,
        out_shape=jax.ShapeDtypeStruct((nr, dr), jnp.float32),
    )(degp2)


def _tc_scale(dinvr, xwr):
    nr, dr = xwr.shape

    def body(dinv_ref, xw_ref, g0_ref):
        g0_ref[...] = xw_ref[...] * dinv_ref[...]

    return pl.pallas_call(
        body,
        out_shape=jax.ShapeDtypeStruct((nr, dr), jnp.float32),
    )(dinvr, xwr)


def _tc_mid(partsr, g0r, dinvr, b0t):
    nr, dr = g0r.shape

    def body(p_ref, g0_ref, dinv_ref, b_ref, g1_ref):
        ssum = p_ref[0, :nr, :] + p_ref[1, :nr, :] + g0_ref[...]
        h0 = jnp.maximum(ssum * dinv_ref[...] + b_ref[...], 0.0)
        g1_ref[...] = h0 * dinv_ref[...]

    return pl.pallas_call(
        body,
        out_shape=jax.ShapeDtypeStruct((nr, dr), jnp.float32),
    )(partsr, g0r, dinvr, b0t)


def _tc_out(partsr, g1r, dinvr, w1r, b1t):
    nr, dr = g1r.shape
    ncr = w1r.shape[1]

    def body(p_ref, g1_ref, dinv_ref, w_ref, b_ref, out_ref):
        ssum = (p_ref[0, :nr, :] + p_ref[1, :nr, :] + g1_ref[...]) * dinv_ref[...]
        out_ref[...] = jnp.dot(ssum, w_ref[...],
                               preferred_element_type=jnp.float32) + b_ref[...]

    return pl.pallas_call(
        body,
        out_shape=jax.ShapeDtypeStruct((nr, ncr), jnp.float32),
    )(partsr, g1r, dinvr, w1r, b1t)


def _block_diag(w, k):
    a, b = w.shape
    eye = jnp.eye(k, dtype=w.dtype)
    return (eye[:, None, :, None] * w[None, :, None, :]).reshape(k * a, k * b)


def kernel(x, edge_index, W0, b0, W1, b1):
    n = x.shape[1]
    e = edge_index.shape[2]
    dh = W0.shape[1]

    x2 = x[0]
    ei = edge_index[0].astype(jnp.int32)

    cpw = -(-e // (_NW * _CH))
    cpw += (-cpw) % 4
    epad = _NW * cpw * _CH
    npad_rows = 240
    nacc = n + npad_rows

    pad = epad - e
    pad_ar = jnp.arange(pad, dtype=jnp.int32)
    pads = jnp.stack([pad_ar % n, n + (pad_ar % npad_rows)])
    edges = jnp.concatenate([ei, pads], axis=1).reshape(2, _NW, cpw, _CH)

    ones_c1 = jnp.ones((_CH,), jnp.float32)
    zeros1 = jnp.zeros((nacc,), jnp.float32)
    zeros2 = jnp.zeros((nacc, dh), jnp.float32)

    pk = 128 // dh
    nr = n // pk
    nc = W1.shape[1]
    xr = x2.reshape(nr, pk * x2.shape[1])
    w0r = _block_diag(W0, pk)
    w1r = _block_diag(W1, pk)
    b0t = jnp.tile(b0, pk).reshape(1, pk * dh)
    b1t = jnp.tile(b1, pk).reshape(1, pk * nc)

    xwr = _tc_matmul(xr, w0r)
    degp = _sc_degree(nacc, cpw)(edges, zeros1, ones_c1)
    dinv1 = _tc_rsqrt(degp.reshape(_NC, nacc // 128, 128))
    dinvr = jnp.repeat(dinv1.reshape(-1)[:n], dh).reshape(nr, pk * dh)
    g0r = _tc_scale(dinvr, xwr)
    g0 = g0r.reshape(n, dh)
    parts1 = _sc_propagate(n, nacc, cpw, dh)(g0, edges, zeros2)
    g1r = _tc_mid(parts1.reshape(_NC, nacc // pk, pk * dh), g0r, dinvr, b0t)
    g1 = g1r.reshape(n, dh)
    parts2 = _sc_propagate(n, nacc, cpw, dh)(g1, edges, zeros2)
    outr = _tc_out(parts2.reshape(_NC, nacc // pk, pk * dh), g1r, dinvr,
                   w1r, b1t)
    return outr.reshape(n, nc)


# speedup vs baseline: 1.0659x; 1.0052x over previous
import functools

import jax
import jax.numpy as jnp
from jax import lax
from jax.experimental import pallas as pl
from jax.experimental.pallas import tpu as pltpu
from jax.experimental.pallas import tpu_sc as plsc

_NC = 2
_NS = 16
_NW = _NC * _NS
_CH = 128


def _mesh():
    return plsc.VectorSubcoreMesh(
        core_axis_name="c", subcore_axis_name="s",
        num_cores=_NC, num_subcores=_NS)


def _sc_degree(nacc, cpw, d):
    seg = nacc // _NS

    @functools.partial(
        pl.kernel, mesh=_mesh(),
        out_type=jax.ShapeDtypeStruct((_NC, nacc, d), jnp.float32),
        compiler_params=pltpu.CompilerParams(use_tc_tiling_on_sc=False),
        scratch_types=[
            pltpu.VMEM((cpw, _CH), jnp.int32),
            pltpu.VMEM((_CH, d), jnp.float32),
            pltpu.VMEM_SHARED((nacc, d), jnp.float32),
            pltpu.SemaphoreType.DMA,
        ],
    )
    def deg_kernel(edges_hbm, zeros_hbm, ones_hbm, out_hbm, didx, ones_v, acc,
                   sem):
        c = lax.axis_index("c")
        s = lax.axis_index("s")
        wid = s * _NC + c
        pltpu.sync_copy(edges_hbm.at[1, wid], didx)
        pltpu.sync_copy(ones_hbm, ones_v)
        pltpu.sync_copy(zeros_hbm.at[pl.ds(s * seg, seg), :],
                        acc.at[pl.ds(s * seg, seg), :])
        plsc.subcore_barrier()

        @pl.loop(0, cpw)
        def _(j):
            pltpu.async_copy(ones_v, acc.at[didx.at[j]], sem, add=True)

        @pl.loop(0, cpw)
        def _(j):
            pltpu.make_async_copy(ones_v, acc.at[didx.at[0]], sem).wait()

        plsc.subcore_barrier()
        pltpu.sync_copy(acc.at[pl.ds(s * seg, seg), :],
                        out_hbm.at[c, pl.ds(s * seg, seg), :])

    return deg_kernel


def _sc_propagate(n, nacc, cpw, d):
    seg = nacc // _NS
    gseg = n // _NS

    @functools.partial(
        pl.kernel, mesh=_mesh(),
        out_type=jax.ShapeDtypeStruct((_NC, nacc, d), jnp.float32),
        compiler_params=pltpu.CompilerParams(use_tc_tiling_on_sc=False),
        scratch_types=[
            pltpu.VMEM((cpw, _CH), jnp.int32),
            pltpu.VMEM((cpw, _CH), jnp.int32),
            pltpu.VMEM((4, _CH, d), jnp.float32),
            pltpu.VMEM_SHARED((nacc, d), jnp.float32),
            pltpu.VMEM_SHARED((n, d), jnp.float32),
            [pltpu.SemaphoreType.DMA] * 4,
            [pltpu.SemaphoreType.DMA] * 4,
        ],
    )
    def prop_kernel(g_hbm, edges_hbm, zeros_hbm, out_hbm,
                    sidx, didx, bufs, acc, g_sh, gsem, ssem):
        c = lax.axis_index("c")
        s = lax.axis_index("s")
        wid = s * _NC + c
        pltpu.sync_copy(edges_hbm.at[0, wid], sidx)
        pltpu.sync_copy(edges_hbm.at[1, wid], didx)
        pltpu.sync_copy(zeros_hbm.at[pl.ds(s * seg, seg), :],
                        acc.at[pl.ds(s * seg, seg), :])
        pltpu.sync_copy(g_hbm.at[pl.ds(s * gseg, gseg), :],
                        g_sh.at[pl.ds(s * gseg, gseg), :])
        plsc.subcore_barrier()

        for b in range(3):
            pltpu.async_copy(g_sh.at[sidx.at[b]], bufs.at[b], gsem[b])

        @pl.loop(0, cpw, step=4)
        def _(j):
            for b in range(4):
                jb = j + b
                ns = (b + 3) % 4

                @pl.when(jb > 0)
                def _():
                    pltpu.make_async_copy(
                        bufs.at[ns], acc.at[didx.at[0]], ssem[ns]).wait()

                @pl.when(jb + 3 < cpw)
                def _():
                    pltpu.async_copy(
                        g_sh.at[sidx.at[jb + 3]], bufs.at[ns], gsem[ns])

                pltpu.make_async_copy(
                    g_sh.at[sidx.at[jb]], bufs.at[b], gsem[b]).wait()
                pltpu.async_copy(
                    bufs.at[b], acc.at[didx.at[jb]], ssem[b], add=True)

        pltpu.make_async_copy(bufs.at[3], acc.at[didx.at[0]], ssem[3]).wait()
        plsc.subcore_barrier()
        pltpu.sync_copy(acc.at[pl.ds(s * seg, seg), :],
                        out_hbm.at[c, pl.ds(s * seg, seg), :])

    return prop_kernel


def _tc_matmul(xr, w0r):
    nr = xr.shape[0]
    dr = w0r.shape[1]

    def body(x_ref, w_ref, xw_ref):
        xw_ref[...] = jnp.dot(x_ref[...], w_ref[...],
                              preferred_element_type=jnp.float32)

    return pl.pallas_call(
        body,
        out_shape=jax.ShapeDtypeStruct((nr, dr), jnp.float32),
    )(xr, w0r)


def _tc_scale(degpr, xwr):
    nr, dr = xwr.shape

    def body(degp_ref, xw_ref, g0_ref, dinv_ref):
        deg = degp_ref[0, :nr, :] + degp_ref[1, :nr, :] + 1.0
        dinv = lax.rsqrt(deg)
        dinv_ref[...] = dinv
        g0_ref[...] = xw_ref[...] * dinv

    return pl.pallas_call(
        body,
        out_shape=(jax.ShapeDtypeStruct((nr, dr), jnp.float32),
                   jax.ShapeDtypeStruct((nr, dr), jnp.float32)),
    )(degpr, xwr)


def _tc_mid(partsr, g0r, dinvr, b0t):
    nr, dr = g0r.shape

    def body(p_ref, g0_ref, dinv_ref, b_ref, g1_ref):
        ssum = p_ref[0, :nr, :] + p_ref[1, :nr, :] + g0_ref[...]
        h0 = jnp.maximum(ssum * dinv_ref[...] + b_ref[...], 0.0)
        g1_ref[...] = h0 * dinv_ref[...]

    return pl.pallas_call(
        body,
        out_shape=jax.ShapeDtypeStruct((nr, dr), jnp.float32),
    )(partsr, g0r, dinvr, b0t)


def _tc_out(partsr, g1r, dinvr, w1r, b1t):
    nr, dr = g1r.shape
    ncr = w1r.shape[1]

    def body(p_ref, g1_ref, dinv_ref, w_ref, b_ref, out_ref):
        ssum = (p_ref[0, :nr, :] + p_ref[1, :nr, :] + g1_ref[...]) * dinv_ref[...]
        out_ref[...] = jnp.dot(ssum, w_ref[...],
                               preferred_element_type=jnp.float32) + b_ref[...]

    return pl.pallas_call(
        body,
        out_shape=jax.ShapeDtypeStruct((nr, ncr), jnp.float32),
    )(partsr, g1r, dinvr, w1r, b1t)


def _block_diag(w, k):
    a, b = w.shape
    eye = jnp.eye(k, dtype=w.dtype)
    return (eye[:, None, :, None] * w[None, :, None, :]).reshape(k * a, k * b)


def kernel(x, edge_index, W0, b0, W1, b1):
    n = x.shape[1]
    e = edge_index.shape[2]
    dh = W0.shape[1]

    x2 = x[0]
    ei = edge_index[0].astype(jnp.int32)

    cpw = -(-e // (_NW * _CH))
    cpw += (-cpw) % 4
    epad = _NW * cpw * _CH
    npad_rows = 240
    nacc = n + npad_rows

    pad = epad - e
    pad_ar = jnp.arange(pad, dtype=jnp.int32)
    pads = jnp.stack([pad_ar % n, n + (pad_ar % npad_rows)])
    edges = jnp.concatenate([ei, pads], axis=1).reshape(2, _NW, cpw, _CH)

    ones_c = jnp.ones((_CH, dh), jnp.float32)
    zeros2 = jnp.zeros((nacc, dh), jnp.float32)

    pk = 128 // dh
    nr = n // pk
    nc = W1.shape[1]
    xr = x2.reshape(nr, pk * x2.shape[1])
    w0r = _block_diag(W0, pk)
    w1r = _block_diag(W1, pk)
    b0t = jnp.tile(b0, pk).reshape(1, pk * dh)
    b1t = jnp.tile(b1, pk).reshape(1, pk * nc)

    xwr = _tc_matmul(xr, w0r)
    degp = _sc_degree(nacc, cpw, dh)(edges, zeros2, ones_c)
    degpr = degp.reshape(_NC, nacc // pk, pk * dh)
    g0r, dinvr = _tc_scale(degpr, xwr)
    g0 = g0r.reshape(n, dh)
    parts1 = _sc_propagate(n, nacc, cpw, dh)(g0, edges, zeros2)
    g1r = _tc_mid(parts1.reshape(_NC, nacc // pk, pk * dh), g0r, dinvr, b0t)
    g1 = g1r.reshape(n, dh)
    parts2 = _sc_propagate(n, nacc, cpw, dh)(g1, edges, zeros2)
    outr = _tc_out(parts2.reshape(_NC, nacc // pk, pk * dh), g1r, dinvr,
                   w1r, b1t)
    return outr.reshape(n, nc)
